# baseline (device time: 48939 ns/iter reference)
import jax
import jax.numpy as jnp
from jax import lax
from jax.experimental import pallas as pl
from jax.experimental.pallas import tpu as pltpu

N_DEV = 4


def kernel(dy, W):
    m, k = dy.shape
    d = W.shape[0]

    def body(dy_ref, w_ref, out_ref, comm_ref, send_sems, recv_sems):
        my_pos = lax.axis_index("i")
        left = (my_pos - 1) % N_DEV
        right = (my_pos + 1) % N_DEV

        barrier_sem = pltpu.get_barrier_semaphore()
        for nbr in [left, right]:
            pl.semaphore_signal(
                barrier_sem, inc=1,
                device_id=(nbr,), device_id_type=pl.DeviceIdType.MESH,
            )
        pl.semaphore_wait(barrier_sem, 2)

        partial = lax.dot_general(
            dy_ref[:, :], w_ref[:, :],
            dimension_numbers=(((1,), (1,)), ((), ())),
            preferred_element_type=jnp.float32,
        )
        out_ref[:, :] = partial
        comm_ref[0, :, :] = partial

        for h in range(N_DEV - 1):
            send_slot = h % 2
            recv_slot = (h + 1) % 2
            rdma = pltpu.make_async_remote_copy(
                src_ref=comm_ref.at[send_slot],
                dst_ref=comm_ref.at[recv_slot],
                send_sem=send_sems.at[send_slot],
                recv_sem=recv_sems.at[recv_slot],
                device_id=(right,),
                device_id_type=pl.DeviceIdType.MESH,
            )
            rdma.start()
            rdma.wait()
            out_ref[:, :] += comm_ref[recv_slot, :, :]

    return pl.pallas_call(
        body,
        out_shape=jax.ShapeDtypeStruct((m, d), jnp.float32),
        in_specs=[
            pl.BlockSpec(memory_space=pltpu.VMEM),
            pl.BlockSpec(memory_space=pltpu.VMEM),
        ],
        out_specs=pl.BlockSpec(memory_space=pltpu.VMEM),
        scratch_shapes=[
            pltpu.VMEM((2, m, d), jnp.float32),
            pltpu.SemaphoreType.DMA((2,)),
            pltpu.SemaphoreType.DMA((2,)),
        ],
        compiler_params=pltpu.CompilerParams(collective_id=0),
    )(dy, W)


# device time: 23536 ns/iter; 2.0793x vs baseline; 2.0793x over previous
import jax
import jax.numpy as jnp
from jax import lax
from jax.experimental import pallas as pl
from jax.experimental.pallas import tpu as pltpu

N_DEV = 4
Q = 128


def kernel(dy, W):
    m, k_dim = dy.shape
    d = W.shape[0]

    def body(dy_ref, w_ref, out_ref, recv_buf, send_sems, recv_sems):
        k = lax.axis_index("i")
        pA = jnp.bitwise_xor(k, 1)
        pB = 3 - k

        barrier_sem = pltpu.get_barrier_semaphore()
        for nbr in [pA, pB]:
            pl.semaphore_signal(
                barrier_sem, inc=1,
                device_id=(nbr,), device_id_type=pl.DeviceIdType.MESH,
            )
        pl.semaphore_wait(barrier_sem, 2)

        out_ref[:, :] = lax.dot_general(
            dy_ref[:, :], w_ref[:, :],
            dimension_numbers=(((1,), (1,)), ((), ())),
            preferred_element_type=jnp.float32,
        )

        in12 = jnp.logical_or(k == 1, k == 2).astype(jnp.int32)
        keep0 = in12 * Q
        send0 = Q - keep0
        keep1 = 2 * Q + (k // 2) * Q
        send1 = 2 * Q + (1 - k // 2) * Q

        def exchange(slot, src_start, partner):
            rdma = pltpu.make_async_remote_copy(
                src_ref=out_ref.at[pl.ds(src_start, Q), :],
                dst_ref=recv_buf.at[slot],
                send_sem=send_sems.at[slot],
                recv_sem=recv_sems.at[slot],
                device_id=(partner,),
                device_id_type=pl.DeviceIdType.MESH,
            )
            rdma.start()
            return rdma

        r0 = exchange(0, send0, pA)
        r1 = exchange(1, send1, pB)

        r0.wait()
        out_ref[pl.ds(keep0, Q), :] += recv_buf[0, :, :]
        r2 = exchange(2, keep0, pB)

        r1.wait()
        out_ref[pl.ds(keep1, Q), :] += recv_buf[1, :, :]
        r3 = exchange(3, keep1, pA)

        r2.wait()
        out_ref[pl.ds(keep0, Q), :] += recv_buf[2, :, :]
        r4 = exchange(4, keep0, pA)

        r3.wait()
        out_ref[pl.ds(keep1, Q), :] += recv_buf[3, :, :]
        r5 = exchange(5, keep1, pB)

        r4.wait()
        out_ref[pl.ds(send0, Q), :] = recv_buf[4, :, :]
        r5.wait()
        out_ref[pl.ds(send1, Q), :] = recv_buf[5, :, :]

    return pl.pallas_call(
        body,
        out_shape=jax.ShapeDtypeStruct((m, d), jnp.float32),
        in_specs=[
            pl.BlockSpec(memory_space=pltpu.VMEM),
            pl.BlockSpec(memory_space=pltpu.VMEM),
        ],
        out_specs=pl.BlockSpec(memory_space=pltpu.VMEM),
        scratch_shapes=[
            pltpu.VMEM((6, Q, d), jnp.float32),
            pltpu.SemaphoreType.DMA((6,)),
            pltpu.SemaphoreType.DMA((6,)),
        ],
        compiler_params=pltpu.CompilerParams(collective_id=0),
    )(dy, W)


# device time: 20985 ns/iter; 2.3321x vs baseline; 1.1216x over previous
import jax
import jax.numpy as jnp
from jax import lax
from jax.experimental import pallas as pl
from jax.experimental.pallas import tpu as pltpu

N_DEV = 4
Q = 128
C = 256


def kernel(dy, W):
    m, k_dim = dy.shape
    d = W.shape[0]

    def body(dy_ref, w_ref, out_ref, recv_buf, send_sems, recv_sems):
        k = lax.axis_index("i")
        pA = jnp.bitwise_xor(k, 1)
        pB = 3 - k

        barrier_sem = pltpu.get_barrier_semaphore()
        for nbr in [pA, pB]:
            pl.semaphore_signal(
                barrier_sem, inc=1,
                device_id=(nbr,), device_id_type=pl.DeviceIdType.MESH,
            )
        pl.semaphore_wait(barrier_sem, 2)

        in12 = jnp.logical_or(k == 1, k == 2).astype(jnp.int32)
        keep0 = in12 * Q
        send0 = Q - keep0
        keep1 = 2 * Q + (k // 2) * Q
        send1 = 2 * Q + (1 - k // 2) * Q

        def sub_gemm(row_start, col):
            out_ref[pl.ds(row_start, Q), pl.ds(col * C, C)] = lax.dot_general(
                dy_ref[pl.ds(row_start, Q), :],
                w_ref[pl.ds(col * C, C), :],
                dimension_numbers=(((1,), (1,)), ((), ())),
                preferred_element_type=jnp.float32,
            )

        def exchange(slot, row_start, col, partner):
            rdma = pltpu.make_async_remote_copy(
                src_ref=out_ref.at[pl.ds(row_start, Q), pl.ds(col * C, C)],
                dst_ref=recv_buf.at[slot],
                send_sem=send_sems.at[slot],
                recv_sem=recv_sems.at[slot],
                device_id=(partner,),
                device_id_type=pl.DeviceIdType.MESH,
            )
            rdma.start()
            return rdma

        ch_col = [0, 1, 0, 1]
        ch_keep = [keep0, keep0, keep1, keep1]
        ch_send = [send0, send0, send1, send1]
        ch_p = [
            [pA, pB, pA], [pA, pB, pA],
            [pB, pA, pB], [pB, pA, pB],
        ]
        order = [0, 2, 1, 3]

        r1 = [None] * 4
        for ch in order:
            sub_gemm(ch_send[ch], ch_col[ch])
            r1[ch] = exchange(ch, ch_send[ch], ch_col[ch], ch_p[ch][0])
        for ch in order:
            sub_gemm(ch_keep[ch], ch_col[ch])

        r2 = [None] * 4
        for ch in order:
            r1[ch].wait()
            out_ref[pl.ds(ch_keep[ch], Q), pl.ds(ch_col[ch] * C, C)] += (
                recv_buf[ch, :, :]
            )
            r2[ch] = exchange(4 + ch, ch_keep[ch], ch_col[ch], ch_p[ch][1])

        r3 = [None] * 4
        for ch in order:
            r2[ch].wait()
            out_ref[pl.ds(ch_keep[ch], Q), pl.ds(ch_col[ch] * C, C)] += (
                recv_buf[4 + ch, :, :]
            )
            r3[ch] = exchange(8 + ch, ch_keep[ch], ch_col[ch], ch_p[ch][2])

        for ch in order:
            r3[ch].wait()
            out_ref[pl.ds(ch_send[ch], Q), pl.ds(ch_col[ch] * C, C)] = (
                recv_buf[8 + ch, :, :]
            )

    return pl.pallas_call(
        body,
        out_shape=jax.ShapeDtypeStruct((m, d), jnp.float32),
        in_specs=[
            pl.BlockSpec(memory_space=pltpu.VMEM),
            pl.BlockSpec(memory_space=pltpu.VMEM),
        ],
        out_specs=pl.BlockSpec(memory_space=pltpu.VMEM),
        scratch_shapes=[
            pltpu.VMEM((12, Q, C), jnp.float32),
            pltpu.SemaphoreType.DMA((12,)),
            pltpu.SemaphoreType.DMA((12,)),
        ],
        compiler_params=pltpu.CompilerParams(collective_id=0),
    )(dy, W)
